# baseline (speedup 1.0000x reference)
import functools

import jax
import jax.numpy as jnp
from jax import lax
from jax.experimental import pallas as pl
from jax.experimental.pallas import tpu as pltpu
from jax.experimental.pallas import tpu_sc as plsc

N_NODES = 10000
N_EDGES = 320000
NC = 2
NS = 16
NW = NC * NS
CHUNK = 128
K_CHUNKS = 80
E_PAD = NW * K_CHUNKS * CHUNK
NBUF = 1
CHUNK_D = 128
K_D = E_PAD // (NW * CHUNK_D)
ACC_ROWS = 10240
STRIPE = ACC_ROWS // NS


def _make_sc_pass(feat, gather):
  mesh = plsc.VectorSubcoreMesh(core_axis_name="c", subcore_axis_name="s")

  nbuf = NBUF if gather else 1
  chunk = CHUNK if gather else CHUNK_D
  k_chunks = K_CHUNKS if gather else K_D
  k_stage = 16 if gather else k_chunks
  scratch = []
  if gather:
    scratch.append(pltpu.VMEM((k_stage, chunk), jnp.int32))
  scratch += [
      pltpu.VMEM((k_stage, chunk), jnp.int32),
  ]
  scratch += [pltpu.VMEM((chunk, feat), jnp.float32)
              for _ in range(nbuf)]
  scratch += [
      pltpu.VMEM_SHARED((ACC_ROWS, feat), jnp.float32),
  ]
  scratch += [pltpu.SemaphoreType.DMA for _ in range(2 * nbuf)]

  def body(*refs):
    if gather:
      src_hbm, dst_hbm, t_hbm, out_hbm = refs[:4]
      refs = refs[4:]
      src_v, dst_v = refs[:2]
      refs = refs[2:]
    else:
      dst_hbm, out_hbm = refs[:2]
      refs = refs[2:]
      dst_v = refs[0]
      refs = refs[1:]
    rows = refs[:nbuf]
    acc = refs[nbuf]
    gsem = refs[nbuf + 1:nbuf + 1 + nbuf]
    ssem = refs[nbuf + 1 + nbuf:]

    c = lax.axis_index("c")
    s = lax.axis_index("s")

    def fill_rows(val):
      vec = jnp.full((16,), val, jnp.float32)
      def fb(i, carry):
        for jj in range(feat // 16):
          rows[0][i, pl.ds(jj * 16, 16)] = vec
        return carry
      lax.fori_loop(0, chunk, fb, 0)

    fill_rows(0.0)
    base = s * STRIPE
    for k in range(STRIPE // chunk):
      pltpu.sync_copy(rows[0], acc.at[pl.ds(base + k * chunk, chunk)])
    plsc.subcore_barrier()

    if not gather:
      fill_rows(1.0)

    if gather:
      def fire_gather(j, b):
        return pltpu.async_copy(t_hbm.at[src_v.at[j]], rows[b], gsem[b])

      def fire_scatter(j, b):
        return pltpu.async_copy(rows[b], acc.at[dst_v.at[j]], ssem[b],
                                add=True)

      for stage in range(k_chunks // k_stage):
        pltpu.sync_copy(dst_hbm.at[c, s, pl.ds(stage * k_stage, k_stage)],
                        dst_v)
        pltpu.sync_copy(src_hbm.at[c, s, pl.ds(stage * k_stage, k_stage)],
                        src_v)

        def chunk_loop(j, carry):
          fire_gather(j, 0).wait()
          fire_scatter(j, 0).wait()
          return carry
        lax.fori_loop(0, k_stage, chunk_loop, 0)
    else:
      pltpu.sync_copy(dst_hbm.at[c, s], dst_v)
      def chunk_body(j, carry):
        pltpu.sync_copy(rows[0], acc.at[dst_v.at[j]], add=True)
        return carry
      lax.fori_loop(0, k_chunks, chunk_body, 0)

    plsc.subcore_barrier()

    pltpu.sync_copy(acc.at[pl.ds(base, STRIPE)],
                    out_hbm.at[c, pl.ds(base, STRIPE)])

  return pl.kernel(
      body,
      out_type=jax.ShapeDtypeStruct((NC, ACC_ROWS, feat), jnp.float32),
      mesh=mesh,
      scratch_types=scratch,
  )


_ROWS = 1000
_GRID = N_NODES // _ROWS


def _row_spec(feat):
  return pl.BlockSpec((_ROWS, feat), lambda i: (i, 0))


def _full_spec(r, cdim):
  return pl.BlockSpec((r, cdim), lambda i: (0, 0))


def _tc_first(x, w1, d0, d1):
  def body(x_ref, w_ref, d0_ref, d1_ref, o_ref):
    dinv = lax.rsqrt(d0_ref[...] + d1_ref[...] + 1.0)
    o_ref[...] = dinv * jnp.dot(x_ref[...], w_ref[...],
                                preferred_element_type=jnp.float32)
  return pl.pallas_call(
      body,
      grid=(_GRID,),
      in_specs=[_row_spec(128), _full_spec(128, 128),
                _row_spec(1), _row_spec(1)],
      out_specs=_row_spec(128),
      out_shape=jax.ShapeDtypeStruct((N_NODES, 128), jnp.float32),
  )(x, w1, d0, d1)


def _tc_mid(p0, p1, t, d0, d1, b, w, fout):
  def body(p0_ref, p1_ref, t_ref, d0_ref, d1_ref, b_ref, w_ref, o_ref):
    dinv = lax.rsqrt(d0_ref[...] + d1_ref[...] + 1.0)
    h = jnp.tanh(dinv * (p0_ref[...] + p1_ref[...] + t_ref[...]) + b_ref[...])
    o_ref[...] = dinv * jnp.dot(h, w_ref[...],
                                preferred_element_type=jnp.float32)
  return pl.pallas_call(
      body,
      grid=(_GRID,),
      in_specs=[_row_spec(128), _row_spec(128), _row_spec(128),
                _row_spec(1), _row_spec(1),
                _full_spec(1, 128), _full_spec(128, fout)],
      out_specs=_row_spec(fout),
      out_shape=jax.ShapeDtypeStruct((N_NODES, fout), jnp.float32),
  )(p0, p1, t, d0, d1, b, w)


def _tc_last(p0, p1, t, d0, d1, b3, wc, bc):
  def body(p0_ref, p1_ref, t_ref, d0_ref, d1_ref, b_ref, wc_ref, bc_ref,
           out_ref, h_ref):
    dinv = lax.rsqrt(d0_ref[...] + d1_ref[...] + 1.0)
    acc = (p0_ref[...] + p1_ref[...] + t_ref[...])[:, :64]
    h = jnp.tanh(dinv * acc + b_ref[...])
    h_ref[...] = h
    out_ref[...] = jnp.dot(h, wc_ref[...],
                           preferred_element_type=jnp.float32) + bc_ref[...]
  return pl.pallas_call(
      body,
      grid=(_GRID,),
      in_specs=[_row_spec(128), _row_spec(128), _row_spec(128),
                _row_spec(1), _row_spec(1),
                _full_spec(1, 64), _full_spec(64, 16), _full_spec(1, 16)],
      out_specs=[_row_spec(16), _row_spec(64)],
      out_shape=[jax.ShapeDtypeStruct((N_NODES, 16), jnp.float32),
                 jax.ShapeDtypeStruct((N_NODES, 64), jnp.float32)],
  )(p0, p1, t, d0, d1, b3, wc, bc)


_sc_deg = _make_sc_pass(16, gather=False)
_sc_agg128 = _make_sc_pass(128, gather=True)


def kernel(x, edge_index, W1, b1, W2, b2, W3, b3, Wc, bc):
  ei = edge_index.astype(jnp.int32)
  npad = E_PAD - N_EDGES
  src = jnp.concatenate([ei[0], jnp.zeros((npad,), jnp.int32)])
  dst = jnp.concatenate([ei[1], jnp.full((npad,), N_NODES, jnp.int32)])
  src_r = src.reshape(NC, NS, K_CHUNKS, CHUNK)
  dst_r = dst.reshape(NC, NS, K_CHUNKS, CHUNK)
  dst_d = dst.reshape(NC, NS, K_D, CHUNK_D)

  deg_parts = _sc_deg(dst_d)
  d0 = deg_parts[0, :N_NODES, 0:1]
  d1 = deg_parts[1, :N_NODES, 0:1]

  b1r = b1.reshape(1, 128)
  b2r = b2.reshape(1, 128)
  b3r = b3.reshape(1, 64)
  bcr = bc.reshape(1, 16)

  t1 = _tc_first(x, W1, d0, d1)
  a1 = _sc_agg128(src_r, dst_r, t1)
  t2 = _tc_mid(a1[0, :N_NODES], a1[1, :N_NODES], t1, d0, d1, b1r, W2, 128)
  a2 = _sc_agg128(src_r, dst_r, t2)
  w3p = jnp.pad(W3, ((0, 0), (0, 64)))
  t3 = _tc_mid(a2[0, :N_NODES], a2[1, :N_NODES], t2, d0, d1, b2r, w3p, 128)
  a3 = _sc_agg128(src_r, dst_r, t3)
  out, h3 = _tc_last(a3[0, :N_NODES], a3[1, :N_NODES], t3, d0, d1,
                     b3r, Wc, bcr)
  return (out, h3)


# speedup vs baseline: 1.0051x; 1.0051x over previous
import functools

import jax
import jax.numpy as jnp
from jax import lax
from jax.experimental import pallas as pl
from jax.experimental.pallas import tpu as pltpu
from jax.experimental.pallas import tpu_sc as plsc

N_NODES = 10000
N_EDGES = 320000
NC = 2
NS = 16
NW = NC * NS
CHUNK = 128
K_CHUNKS = 80
E_PAD = NW * K_CHUNKS * CHUNK
NBUF = 1
CHUNK_D = 128
K_D = E_PAD // (NW * CHUNK_D)
ACC_ROWS = 10240
STRIPE = ACC_ROWS // NS


def _make_sc_pass(feat, gather):
  mesh = plsc.VectorSubcoreMesh(core_axis_name="c", subcore_axis_name="s")

  nbuf = NBUF if gather else 1
  chunk = CHUNK if gather else CHUNK_D
  k_chunks = K_CHUNKS if gather else K_D
  k_stage = k_chunks
  scratch = []
  if gather:
    scratch.append(pltpu.VMEM((k_stage, chunk), jnp.int32))
  scratch += [
      pltpu.VMEM((k_stage, chunk), jnp.int32),
  ]
  scratch += [pltpu.VMEM((chunk, feat), jnp.float32)
              for _ in range(nbuf)]
  scratch += [
      pltpu.VMEM_SHARED((ACC_ROWS, feat), jnp.float32),
  ]
  scratch += [pltpu.SemaphoreType.DMA for _ in range(2 * nbuf)]

  def body(*refs):
    if gather:
      src_hbm, dst_hbm, t_hbm, out_hbm = refs[:4]
      refs = refs[4:]
      src_v, dst_v = refs[:2]
      refs = refs[2:]
    else:
      dst_hbm, out_hbm = refs[:2]
      refs = refs[2:]
      dst_v = refs[0]
      refs = refs[1:]
    rows = refs[:nbuf]
    acc = refs[nbuf]
    gsem = refs[nbuf + 1:nbuf + 1 + nbuf]
    ssem = refs[nbuf + 1 + nbuf:]

    c = lax.axis_index("c")
    s = lax.axis_index("s")

    def fill_rows(val):
      vec = jnp.full((16,), val, jnp.float32)
      def fb(i, carry):
        for jj in range(feat // 16):
          rows[0][i, pl.ds(jj * 16, 16)] = vec
        return carry
      lax.fori_loop(0, chunk, fb, 0)

    fill_rows(0.0)
    base = s * STRIPE
    for k in range(STRIPE // chunk):
      pltpu.sync_copy(rows[0], acc.at[pl.ds(base + k * chunk, chunk)])
    plsc.subcore_barrier()

    if not gather:
      fill_rows(1.0)

    if gather:
      def fire_gather(j, b):
        return pltpu.async_copy(t_hbm.at[src_v.at[j]], rows[b], gsem[b])

      def fire_scatter(j, b):
        return pltpu.async_copy(rows[b], acc.at[dst_v.at[j]], ssem[b],
                                add=True)

      for stage in range(k_chunks // k_stage):
        pltpu.sync_copy(dst_hbm.at[c, s, pl.ds(stage * k_stage, k_stage)],
                        dst_v)
        pltpu.sync_copy(src_hbm.at[c, s, pl.ds(stage * k_stage, k_stage)],
                        src_v)

        def chunk_loop(j, carry):
          fire_gather(j, 0).wait()
          fire_scatter(j, 0).wait()
          return carry
        lax.fori_loop(0, k_stage, chunk_loop, 0)
    else:
      pltpu.sync_copy(dst_hbm.at[c, s], dst_v)
      def chunk_body(j, carry):
        pltpu.sync_copy(rows[0], acc.at[dst_v.at[j]], add=True)
        return carry
      lax.fori_loop(0, k_chunks, chunk_body, 0)

    plsc.subcore_barrier()

    for k in range(STRIPE // chunk):
      pltpu.sync_copy(acc.at[pl.ds(base + k * chunk, chunk)], rows[0])
      pltpu.sync_copy(rows[0], out_hbm.at[c, pl.ds(base + k * chunk, chunk)])

  return pl.kernel(
      body,
      out_type=jax.ShapeDtypeStruct((NC, ACC_ROWS, feat), jnp.float32),
      mesh=mesh,
      scratch_types=scratch,
  )


_ROWS = 1000
_GRID = N_NODES // _ROWS


def _row_spec(feat):
  return pl.BlockSpec((_ROWS, feat), lambda i: (i, 0))


def _full_spec(r, cdim):
  return pl.BlockSpec((r, cdim), lambda i: (0, 0))


def _tc_first(x, w1, d0, d1):
  def body(x_ref, w_ref, d0_ref, d1_ref, o_ref):
    dinv = lax.rsqrt(d0_ref[...] + d1_ref[...] + 1.0)
    o_ref[...] = dinv * jnp.dot(x_ref[...], w_ref[...],
                                preferred_element_type=jnp.float32)
  return pl.pallas_call(
      body,
      grid=(_GRID,),
      in_specs=[_row_spec(128), _full_spec(128, 128),
                _row_spec(1), _row_spec(1)],
      out_specs=_row_spec(128),
      out_shape=jax.ShapeDtypeStruct((N_NODES, 128), jnp.float32),
  )(x, w1, d0, d1)


def _tc_mid(p0, p1, t, d0, d1, b, w, fout):
  def body(p0_ref, p1_ref, t_ref, d0_ref, d1_ref, b_ref, w_ref, o_ref):
    dinv = lax.rsqrt(d0_ref[...] + d1_ref[...] + 1.0)
    h = jnp.tanh(dinv * (p0_ref[...] + p1_ref[...] + t_ref[...]) + b_ref[...])
    o_ref[...] = dinv * jnp.dot(h, w_ref[...],
                                preferred_element_type=jnp.float32)
  return pl.pallas_call(
      body,
      grid=(_GRID,),
      in_specs=[_row_spec(128), _row_spec(128), _row_spec(128),
                _row_spec(1), _row_spec(1),
                _full_spec(1, 128), _full_spec(128, fout)],
      out_specs=_row_spec(fout),
      out_shape=jax.ShapeDtypeStruct((N_NODES, fout), jnp.float32),
  )(p0, p1, t, d0, d1, b, w)


def _tc_last(p0, p1, t, d0, d1, b3, wc, bc):
  def body(p0_ref, p1_ref, t_ref, d0_ref, d1_ref, b_ref, wc_ref, bc_ref,
           out_ref, h_ref):
    dinv = lax.rsqrt(d0_ref[...] + d1_ref[...] + 1.0)
    acc = (p0_ref[...] + p1_ref[...] + t_ref[...])[:, :64]
    h = jnp.tanh(dinv * acc + b_ref[...])
    h_ref[...] = h
    out_ref[...] = jnp.dot(h, wc_ref[...],
                           preferred_element_type=jnp.float32) + bc_ref[...]
  return pl.pallas_call(
      body,
      grid=(_GRID,),
      in_specs=[_row_spec(128), _row_spec(128), _row_spec(128),
                _row_spec(1), _row_spec(1),
                _full_spec(1, 64), _full_spec(64, 16), _full_spec(1, 16)],
      out_specs=[_row_spec(16), _row_spec(64)],
      out_shape=[jax.ShapeDtypeStruct((N_NODES, 16), jnp.float32),
                 jax.ShapeDtypeStruct((N_NODES, 64), jnp.float32)],
  )(p0, p1, t, d0, d1, b3, wc, bc)


_sc_deg = _make_sc_pass(16, gather=False)
_sc_agg128 = _make_sc_pass(128, gather=True)


def kernel(x, edge_index, W1, b1, W2, b2, W3, b3, Wc, bc):
  ei = edge_index.astype(jnp.int32)
  npad = E_PAD - N_EDGES
  src = jnp.concatenate([ei[0], jnp.zeros((npad,), jnp.int32)])
  dst = jnp.concatenate([ei[1], jnp.full((npad,), N_NODES, jnp.int32)])
  src_r = src.reshape(NC, NS, K_CHUNKS, CHUNK)
  dst_r = dst.reshape(NC, NS, K_CHUNKS, CHUNK)
  dst_d = dst.reshape(NC, NS, K_D, CHUNK_D)

  deg_parts = _sc_deg(dst_d)
  d0 = deg_parts[0, :N_NODES, 0:1]
  d1 = deg_parts[1, :N_NODES, 0:1]

  b1r = b1.reshape(1, 128)
  b2r = b2.reshape(1, 128)
  b3r = b3.reshape(1, 64)
  bcr = bc.reshape(1, 16)

  t1 = _tc_first(x, W1, d0, d1)
  a1 = _sc_agg128(src_r, dst_r, t1)
  t2 = _tc_mid(a1[0, :N_NODES], a1[1, :N_NODES], t1, d0, d1, b1r, W2, 128)
  a2 = _sc_agg128(src_r, dst_r, t2)
  w3p = jnp.pad(W3, ((0, 0), (0, 64)))
  t3 = _tc_mid(a2[0, :N_NODES], a2[1, :N_NODES], t2, d0, d1, b2r, w3p, 128)
  a3 = _sc_agg128(src_r, dst_r, t3)
  out, h3 = _tc_last(a3[0, :N_NODES], a3[1, :N_NODES], t3, d0, d1,
                     b3r, Wc, bcr)
  return (out, h3)


# speedup vs baseline: 1.0052x; 1.0001x over previous
import functools

import jax
import jax.numpy as jnp
from jax import lax
from jax.experimental import pallas as pl
from jax.experimental.pallas import tpu as pltpu
from jax.experimental.pallas import tpu_sc as plsc

N_NODES = 10000
N_EDGES = 320000
NC = 2
NS = 16
NW = NC * NS
CHUNK = 128
K_CHUNKS = 80
E_PAD = NW * K_CHUNKS * CHUNK
NBUF = 1
CHUNK_D = 128
K_D = E_PAD // (NW * CHUNK_D)
ACC_ROWS = 10240
STRIPE = ACC_ROWS // NS


def _make_sc_pass(feat, gather):
  mesh = plsc.VectorSubcoreMesh(core_axis_name="c", subcore_axis_name="s")

  nbuf = NBUF if gather else 1
  chunk = CHUNK if gather else CHUNK_D
  k_chunks = K_CHUNKS if gather else K_D
  k_stage = k_chunks
  scratch = []
  if gather:
    scratch.append(pltpu.VMEM((k_stage, chunk), jnp.int32))
  scratch += [
      pltpu.VMEM((k_stage, chunk), jnp.int32),
  ]
  scratch += [pltpu.VMEM((chunk, feat), jnp.float32)
              for _ in range(nbuf)]
  scratch += [
      pltpu.VMEM_SHARED((ACC_ROWS, feat), jnp.float32),
  ]
  scratch += [pltpu.SemaphoreType.DMA for _ in range(2 * nbuf)]

  def body(*refs):
    if gather:
      src_hbm, dst_hbm, t_hbm, out_hbm = refs[:4]
      refs = refs[4:]
      src_v, dst_v = refs[:2]
      refs = refs[2:]
    else:
      dst_hbm, out_hbm = refs[:2]
      refs = refs[2:]
      dst_v = refs[0]
      refs = refs[1:]
    rows = refs[:nbuf]
    acc = refs[nbuf]
    gsem = refs[nbuf + 1:nbuf + 1 + nbuf]
    ssem = refs[nbuf + 1 + nbuf:]

    c = lax.axis_index("c")
    s = lax.axis_index("s")

    def fill_rows(val):
      vec = jnp.full((16,), val, jnp.float32)
      def fb(i, carry):
        for jj in range(feat // 16):
          rows[0][i, pl.ds(jj * 16, 16)] = vec
        return carry
      lax.fori_loop(0, chunk, fb, 0)

    fill_rows(0.0)
    base = s * STRIPE
    for k in range(STRIPE // chunk):
      pltpu.sync_copy(rows[0], acc.at[pl.ds(base + k * chunk, chunk)])
    plsc.subcore_barrier()

    if not gather:
      fill_rows(1.0)

    if gather:
      def fire_gather(j, b):
        return pltpu.async_copy(t_hbm.at[src_v.at[j]], rows[b], gsem[b])

      def fire_scatter(j, b):
        return pltpu.async_copy(rows[b], acc.at[dst_v.at[j]], ssem[b],
                                add=True)

      for stage in range(k_chunks // k_stage):
        pltpu.sync_copy(dst_hbm.at[c, s, pl.ds(stage * k_stage, k_stage)],
                        dst_v)
        pltpu.sync_copy(src_hbm.at[c, s, pl.ds(stage * k_stage, k_stage)],
                        src_v)

        def chunk_loop(j, carry):
          fire_gather(j, 0).wait()
          pltpu.sync_copy(rows[0], acc.at[dst_v.at[j]], add=True)
          return carry
        lax.fori_loop(0, k_stage, chunk_loop, 0)
    else:
      pltpu.sync_copy(dst_hbm.at[c, s], dst_v)
      def chunk_body(j, carry):
        pltpu.sync_copy(rows[0], acc.at[dst_v.at[j]], add=True)
        return carry
      lax.fori_loop(0, k_chunks, chunk_body, 0)

    plsc.subcore_barrier()

    for k in range(STRIPE // chunk):
      pltpu.sync_copy(acc.at[pl.ds(base + k * chunk, chunk)], rows[0])
      pltpu.sync_copy(rows[0], out_hbm.at[c, pl.ds(base + k * chunk, chunk)])

  return pl.kernel(
      body,
      out_type=jax.ShapeDtypeStruct((NC, ACC_ROWS, feat), jnp.float32),
      mesh=mesh,
      scratch_types=scratch,
  )


_ROWS = 1000
_GRID = N_NODES // _ROWS


def _row_spec(feat):
  return pl.BlockSpec((_ROWS, feat), lambda i: (i, 0))


def _full_spec(r, cdim):
  return pl.BlockSpec((r, cdim), lambda i: (0, 0))


def _tc_first(x, w1, d0, d1):
  def body(x_ref, w_ref, d0_ref, d1_ref, o_ref):
    dinv = lax.rsqrt(d0_ref[...] + d1_ref[...] + 1.0)
    o_ref[...] = dinv * jnp.dot(x_ref[...], w_ref[...],
                                preferred_element_type=jnp.float32)
  return pl.pallas_call(
      body,
      grid=(_GRID,),
      in_specs=[_row_spec(128), _full_spec(128, 128),
                _row_spec(1), _row_spec(1)],
      out_specs=_row_spec(128),
      out_shape=jax.ShapeDtypeStruct((N_NODES, 128), jnp.float32),
  )(x, w1, d0, d1)


def _tc_mid(p0, p1, t, d0, d1, b, w, fout):
  def body(p0_ref, p1_ref, t_ref, d0_ref, d1_ref, b_ref, w_ref, o_ref):
    dinv = lax.rsqrt(d0_ref[...] + d1_ref[...] + 1.0)
    h = jnp.tanh(dinv * (p0_ref[...] + p1_ref[...] + t_ref[...]) + b_ref[...])
    o_ref[...] = dinv * jnp.dot(h, w_ref[...],
                                preferred_element_type=jnp.float32)
  return pl.pallas_call(
      body,
      grid=(_GRID,),
      in_specs=[_row_spec(128), _row_spec(128), _row_spec(128),
                _row_spec(1), _row_spec(1),
                _full_spec(1, 128), _full_spec(128, fout)],
      out_specs=_row_spec(fout),
      out_shape=jax.ShapeDtypeStruct((N_NODES, fout), jnp.float32),
  )(p0, p1, t, d0, d1, b, w)


def _tc_last(p0, p1, t, d0, d1, b3, wc, bc):
  def body(p0_ref, p1_ref, t_ref, d0_ref, d1_ref, b_ref, wc_ref, bc_ref,
           out_ref, h_ref):
    dinv = lax.rsqrt(d0_ref[...] + d1_ref[...] + 1.0)
    acc = (p0_ref[...] + p1_ref[...] + t_ref[...])[:, :64]
    h = jnp.tanh(dinv * acc + b_ref[...])
    h_ref[...] = h
    out_ref[...] = jnp.dot(h, wc_ref[...],
                           preferred_element_type=jnp.float32) + bc_ref[...]
  return pl.pallas_call(
      body,
      grid=(_GRID,),
      in_specs=[_row_spec(128), _row_spec(128), _row_spec(128),
                _row_spec(1), _row_spec(1),
                _full_spec(1, 64), _full_spec(64, 16), _full_spec(1, 16)],
      out_specs=[_row_spec(16), _row_spec(64)],
      out_shape=[jax.ShapeDtypeStruct((N_NODES, 16), jnp.float32),
                 jax.ShapeDtypeStruct((N_NODES, 64), jnp.float32)],
  )(p0, p1, t, d0, d1, b3, wc, bc)


_sc_deg = _make_sc_pass(16, gather=False)
_sc_agg128 = _make_sc_pass(128, gather=True)


def kernel(x, edge_index, W1, b1, W2, b2, W3, b3, Wc, bc):
  ei = edge_index.astype(jnp.int32)
  npad = E_PAD - N_EDGES
  src = jnp.concatenate([ei[0], jnp.zeros((npad,), jnp.int32)])
  dst = jnp.concatenate([ei[1], jnp.full((npad,), N_NODES, jnp.int32)])
  src_r = src.reshape(NC, NS, K_CHUNKS, CHUNK)
  dst_r = dst.reshape(NC, NS, K_CHUNKS, CHUNK)
  dst_d = dst.reshape(NC, NS, K_D, CHUNK_D)

  deg_parts = _sc_deg(dst_d)
  d0 = deg_parts[0, :N_NODES, 0:1]
  d1 = deg_parts[1, :N_NODES, 0:1]

  b1r = b1.reshape(1, 128)
  b2r = b2.reshape(1, 128)
  b3r = b3.reshape(1, 64)
  bcr = bc.reshape(1, 16)

  t1 = _tc_first(x, W1, d0, d1)
  a1 = _sc_agg128(src_r, dst_r, t1)
  t2 = _tc_mid(a1[0, :N_NODES], a1[1, :N_NODES], t1, d0, d1, b1r, W2, 128)
  a2 = _sc_agg128(src_r, dst_r, t2)
  w3p = jnp.pad(W3, ((0, 0), (0, 64)))
  t3 = _tc_mid(a2[0, :N_NODES], a2[1, :N_NODES], t2, d0, d1, b2r, w3p, 128)
  a3 = _sc_agg128(src_r, dst_r, t3)
  out, h3 = _tc_last(a3[0, :N_NODES], a3[1, :N_NODES], t3, d0, d1,
                     b3r, Wc, bcr)
  return (out, h3)


# speedup vs baseline: 1.0598x; 1.0543x over previous
import functools

import jax
import jax.numpy as jnp
from jax import lax
from jax.experimental import pallas as pl
from jax.experimental.pallas import tpu as pltpu
from jax.experimental.pallas import tpu_sc as plsc

N_NODES = 10000
N_EDGES = 320000
NC = 2
NS = 16
NW = NC * NS
CHUNK = 128
K_TILE = 160
K_A = 96
K_STG = 32
E_PAD = NS * K_TILE * CHUNK
NBUF = 1
CHUNK_D = 128
K_D = E_PAD // (NW * CHUNK_D)
ACC_ROWS = 10240
STRIPE = ACC_ROWS // NS


def _make_sc_pass(feat, gather):
  mesh = plsc.VectorSubcoreMesh(core_axis_name="c", subcore_axis_name="s")

  nbuf = NBUF if gather else 1
  chunk = CHUNK if gather else CHUNK_D
  k_stage = K_STG if gather else K_D
  scratch = []
  if gather:
    scratch.append(pltpu.VMEM((k_stage, chunk), jnp.int32))
  scratch += [
      pltpu.VMEM((k_stage, chunk), jnp.int32),
  ]
  scratch += [pltpu.VMEM((chunk, feat), jnp.float32)
              for _ in range(nbuf)]
  scratch += [
      pltpu.VMEM_SHARED((ACC_ROWS, feat), jnp.float32),
  ]
  scratch += [pltpu.SemaphoreType.DMA for _ in range(2 * nbuf)]

  def body(*refs):
    if gather:
      src_hbm, dst_hbm, t_hbm, out_hbm = refs[:4]
      refs = refs[4:]
      src_v, dst_v = refs[:2]
      refs = refs[2:]
    else:
      dst_hbm, out_hbm = refs[:2]
      refs = refs[2:]
      dst_v = refs[0]
      refs = refs[1:]
    rows = refs[:nbuf]
    acc = refs[nbuf]
    gsem = refs[nbuf + 1:nbuf + 1 + nbuf]
    ssem = refs[nbuf + 1 + nbuf:]

    c = lax.axis_index("c")
    s = lax.axis_index("s")

    def fill_rows(val):
      vec = jnp.full((16,), val, jnp.float32)
      def fb(i, carry):
        for jj in range(feat // 16):
          rows[0][i, pl.ds(jj * 16, 16)] = vec
        return carry
      lax.fori_loop(0, chunk, fb, 0)

    fill_rows(0.0)
    base = s * STRIPE
    for k in range(STRIPE // chunk):
      pltpu.sync_copy(rows[0], acc.at[pl.ds(base + k * chunk, chunk)])
    plsc.subcore_barrier()

    if not gather:
      fill_rows(1.0)

    if gather:
      def fire_gather(j, b):
        return pltpu.async_copy(t_hbm.at[src_v.at[j]], rows[b], gsem[b])

      def fire_scatter(j, b):
        return pltpu.async_copy(rows[b], acc.at[dst_v.at[j]], ssem[b],
                                add=True)

      span0 = lax.select(c == 0, 0, K_A)
      nstages = lax.select(c == 0, K_A // K_STG, (K_TILE - K_A) // K_STG)

      def stage_body(stage, carry):
        off = span0 + stage * K_STG
        pltpu.sync_copy(dst_hbm.at[s, pl.ds(off, K_STG)], dst_v)
        pltpu.sync_copy(src_hbm.at[s, pl.ds(off, K_STG)], src_v)

        def chunk_loop(j, carry2):
          fire_gather(j, 0).wait()
          pltpu.sync_copy(rows[0], acc.at[dst_v.at[j]], add=True)
          return carry2
        lax.fori_loop(0, k_stage, chunk_loop, 0)
        return carry
      lax.fori_loop(0, nstages, stage_body, 0)
    else:
      pltpu.sync_copy(dst_hbm.at[c, s], dst_v)
      def chunk_body(j, carry):
        pltpu.sync_copy(rows[0], acc.at[dst_v.at[j]], add=True)
        return carry
      lax.fori_loop(0, K_D, chunk_body, 0)

    plsc.subcore_barrier()

    for k in range(STRIPE // chunk):
      pltpu.sync_copy(acc.at[pl.ds(base + k * chunk, chunk)], rows[0])
      pltpu.sync_copy(rows[0], out_hbm.at[c, pl.ds(base + k * chunk, chunk)])

  return pl.kernel(
      body,
      out_type=jax.ShapeDtypeStruct((NC, ACC_ROWS, feat), jnp.float32),
      mesh=mesh,
      scratch_types=scratch,
  )


_ROWS = 1000
_GRID = N_NODES // _ROWS


def _row_spec(feat):
  return pl.BlockSpec((_ROWS, feat), lambda i: (i, 0))


def _full_spec(r, cdim):
  return pl.BlockSpec((r, cdim), lambda i: (0, 0))


def _tc_first(x, w1, d0, d1):
  def body(x_ref, w_ref, d0_ref, d1_ref, o_ref):
    dinv = lax.rsqrt(d0_ref[...] + d1_ref[...] + 1.0)
    o_ref[...] = dinv * jnp.dot(x_ref[...], w_ref[...],
                                preferred_element_type=jnp.float32)
  return pl.pallas_call(
      body,
      grid=(_GRID,),
      in_specs=[_row_spec(128), _full_spec(128, 128),
                _row_spec(1), _row_spec(1)],
      out_specs=_row_spec(128),
      out_shape=jax.ShapeDtypeStruct((N_NODES, 128), jnp.float32),
  )(x, w1, d0, d1)


def _tc_mid(p0, p1, t, d0, d1, b, w, fout):
  def body(p0_ref, p1_ref, t_ref, d0_ref, d1_ref, b_ref, w_ref, o_ref):
    dinv = lax.rsqrt(d0_ref[...] + d1_ref[...] + 1.0)
    h = jnp.tanh(dinv * (p0_ref[...] + p1_ref[...] + t_ref[...]) + b_ref[...])
    o_ref[...] = dinv * jnp.dot(h, w_ref[...],
                                preferred_element_type=jnp.float32)
  return pl.pallas_call(
      body,
      grid=(_GRID,),
      in_specs=[_row_spec(128), _row_spec(128), _row_spec(128),
                _row_spec(1), _row_spec(1),
                _full_spec(1, 128), _full_spec(128, fout)],
      out_specs=_row_spec(fout),
      out_shape=jax.ShapeDtypeStruct((N_NODES, fout), jnp.float32),
  )(p0, p1, t, d0, d1, b, w)


def _tc_last(p0, p1, t, d0, d1, b3, wc, bc):
  def body(p0_ref, p1_ref, t_ref, d0_ref, d1_ref, b_ref, wc_ref, bc_ref,
           out_ref, h_ref):
    dinv = lax.rsqrt(d0_ref[...] + d1_ref[...] + 1.0)
    acc = (p0_ref[...] + p1_ref[...] + t_ref[...])[:, :64]
    h = jnp.tanh(dinv * acc + b_ref[...])
    h_ref[...] = h
    out_ref[...] = jnp.dot(h, wc_ref[...],
                           preferred_element_type=jnp.float32) + bc_ref[...]
  return pl.pallas_call(
      body,
      grid=(_GRID,),
      in_specs=[_row_spec(128), _row_spec(128), _row_spec(128),
                _row_spec(1), _row_spec(1),
                _full_spec(1, 64), _full_spec(64, 16), _full_spec(1, 16)],
      out_specs=[_row_spec(16), _row_spec(64)],
      out_shape=[jax.ShapeDtypeStruct((N_NODES, 16), jnp.float32),
                 jax.ShapeDtypeStruct((N_NODES, 64), jnp.float32)],
  )(p0, p1, t, d0, d1, b3, wc, bc)


_sc_deg = _make_sc_pass(16, gather=False)
_sc_agg128 = _make_sc_pass(128, gather=True)


def kernel(x, edge_index, W1, b1, W2, b2, W3, b3, Wc, bc):
  ei = edge_index.astype(jnp.int32)
  npad = E_PAD - N_EDGES
  src = jnp.concatenate([ei[0], jnp.zeros((npad,), jnp.int32)])
  dst = jnp.concatenate([ei[1], jnp.full((npad,), N_NODES, jnp.int32)])
  src_r = src.reshape(NS, K_TILE, CHUNK)
  dst_r = dst.reshape(NS, K_TILE, CHUNK)
  dst_d = dst.reshape(NC, NS, K_D, CHUNK_D)

  deg_parts = _sc_deg(dst_d)
  d0 = deg_parts[0, :N_NODES, 0:1]
  d1 = deg_parts[1, :N_NODES, 0:1]

  b1r = b1.reshape(1, 128)
  b2r = b2.reshape(1, 128)
  b3r = b3.reshape(1, 64)
  bcr = bc.reshape(1, 16)

  t1 = _tc_first(x, W1, d0, d1)
  a1 = _sc_agg128(src_r, dst_r, t1)
  t2 = _tc_mid(a1[0, :N_NODES], a1[1, :N_NODES], t1, d0, d1, b1r, W2, 128)
  a2 = _sc_agg128(src_r, dst_r, t2)
  w3p = jnp.pad(W3, ((0, 0), (0, 64)))
  t3 = _tc_mid(a2[0, :N_NODES], a2[1, :N_NODES], t2, d0, d1, b2r, w3p, 128)
  a3 = _sc_agg128(src_r, dst_r, t3)
  out, h3 = _tc_last(a3[0, :N_NODES], a3[1, :N_NODES], t3, d0, d1,
                     b3r, Wc, bcr)
  return (out, h3)


# speedup vs baseline: 1.1147x; 1.0518x over previous
import functools

import jax
import jax.numpy as jnp
from jax import lax
from jax.experimental import pallas as pl
from jax.experimental.pallas import tpu as pltpu
from jax.experimental.pallas import tpu_sc as plsc

N_NODES = 10000
N_EDGES = 320000
NC = 2
NS = 16
NW = NC * NS
CHUNK = 128
K_TILE = 160
K_A = 96
K_STG = 16
E_PAD = NS * K_TILE * CHUNK
NBUF = 2
CHUNK_D = 128
K_D = E_PAD // (NW * CHUNK_D)
ACC_ROWS = 10240
STRIPE = ACC_ROWS // NS


def _make_sc_pass(feat, gather):
  mesh = plsc.VectorSubcoreMesh(core_axis_name="c", subcore_axis_name="s")

  nbuf = NBUF if gather else 1
  chunk = CHUNK if gather else CHUNK_D
  k_stage = K_STG if gather else K_D
  scratch = []
  if gather:
    scratch.append(pltpu.VMEM((k_stage, chunk), jnp.int32))
  scratch += [
      pltpu.VMEM((k_stage, chunk), jnp.int32),
  ]
  scratch += [pltpu.VMEM((chunk, feat), jnp.float32)
              for _ in range(nbuf)]
  scratch += [
      pltpu.VMEM_SHARED((ACC_ROWS, feat), jnp.float32),
  ]
  scratch += [pltpu.SemaphoreType.DMA for _ in range(2 * nbuf)]

  def body(*refs):
    if gather:
      src_hbm, dst_hbm, t_hbm, out_hbm = refs[:4]
      refs = refs[4:]
      src_v, dst_v = refs[:2]
      refs = refs[2:]
    else:
      dst_hbm, out_hbm = refs[:2]
      refs = refs[2:]
      dst_v = refs[0]
      refs = refs[1:]
    rows = refs[:nbuf]
    acc = refs[nbuf]
    gsem = refs[nbuf + 1:nbuf + 1 + nbuf]
    ssem = refs[nbuf + 1 + nbuf:]

    c = lax.axis_index("c")
    s = lax.axis_index("s")

    def fill_rows(val):
      vec = jnp.full((16,), val, jnp.float32)
      def fb(i, carry):
        for jj in range(feat // 16):
          rows[0][i, pl.ds(jj * 16, 16)] = vec
        return carry
      lax.fori_loop(0, chunk, fb, 0)

    fill_rows(0.0)
    base = s * STRIPE
    for k in range(STRIPE // chunk):
      pltpu.sync_copy(rows[0], acc.at[pl.ds(base + k * chunk, chunk)])
    plsc.subcore_barrier()

    if not gather:
      fill_rows(1.0)

    if gather:
      def fire_gather(j, b):
        return pltpu.async_copy(t_hbm.at[src_v.at[j]], rows[b], gsem[b])

      def fire_scatter(j, b):
        return pltpu.async_copy(rows[b], acc.at[dst_v.at[j]], ssem[b],
                                add=True)

      span0 = lax.select(c == 0, 0, K_A)
      nstages = lax.select(c == 0, K_A // K_STG, (K_TILE - K_A) // K_STG)

      def stage_body(stage, carry):
        off = span0 + stage * K_STG
        pltpu.sync_copy(dst_hbm.at[s, pl.ds(off, K_STG)], dst_v)
        pltpu.sync_copy(src_hbm.at[s, pl.ds(off, K_STG)], src_v)

        gd = [None] * K_STG
        sd = [None] * K_STG
        gd[0] = fire_gather(0, 0)
        gd[1] = fire_gather(1, 1)
        for j in range(K_STG):
          gd[j].wait()
          if j >= 1:
            sd[j - 1].wait()
          sd[j] = fire_scatter(j, j % 2)
          if j >= 1 and j + 1 < K_STG:
            gd[j + 1] = fire_gather(j + 1, (j + 1) % 2)
        sd[K_STG - 1].wait()
        return carry
      lax.fori_loop(0, nstages, stage_body, 0)
    else:
      pltpu.sync_copy(dst_hbm.at[c, s], dst_v)
      def chunk_body(j, carry):
        pltpu.sync_copy(rows[0], acc.at[dst_v.at[j]], add=True)
        return carry
      lax.fori_loop(0, K_D, chunk_body, 0)

    plsc.subcore_barrier()

    for k in range(STRIPE // chunk):
      pltpu.sync_copy(acc.at[pl.ds(base + k * chunk, chunk)], rows[0])
      pltpu.sync_copy(rows[0], out_hbm.at[c, pl.ds(base + k * chunk, chunk)])

  return pl.kernel(
      body,
      out_type=jax.ShapeDtypeStruct((NC, ACC_ROWS, feat), jnp.float32),
      mesh=mesh,
      scratch_types=scratch,
  )


_ROWS = 1000
_GRID = N_NODES // _ROWS


def _row_spec(feat):
  return pl.BlockSpec((_ROWS, feat), lambda i: (i, 0))


def _full_spec(r, cdim):
  return pl.BlockSpec((r, cdim), lambda i: (0, 0))


def _tc_first(x, w1, d0, d1):
  def body(x_ref, w_ref, d0_ref, d1_ref, o_ref):
    dinv = lax.rsqrt(d0_ref[...] + d1_ref[...] + 1.0)
    o_ref[...] = dinv * jnp.dot(x_ref[...], w_ref[...],
                                preferred_element_type=jnp.float32)
  return pl.pallas_call(
      body,
      grid=(_GRID,),
      in_specs=[_row_spec(128), _full_spec(128, 128),
                _row_spec(1), _row_spec(1)],
      out_specs=_row_spec(128),
      out_shape=jax.ShapeDtypeStruct((N_NODES, 128), jnp.float32),
  )(x, w1, d0, d1)


def _tc_mid(p0, p1, t, d0, d1, b, w, fout):
  def body(p0_ref, p1_ref, t_ref, d0_ref, d1_ref, b_ref, w_ref, o_ref):
    dinv = lax.rsqrt(d0_ref[...] + d1_ref[...] + 1.0)
    h = jnp.tanh(dinv * (p0_ref[...] + p1_ref[...] + t_ref[...]) + b_ref[...])
    o_ref[...] = dinv * jnp.dot(h, w_ref[...],
                                preferred_element_type=jnp.float32)
  return pl.pallas_call(
      body,
      grid=(_GRID,),
      in_specs=[_row_spec(128), _row_spec(128), _row_spec(128),
                _row_spec(1), _row_spec(1),
                _full_spec(1, 128), _full_spec(128, fout)],
      out_specs=_row_spec(fout),
      out_shape=jax.ShapeDtypeStruct((N_NODES, fout), jnp.float32),
  )(p0, p1, t, d0, d1, b, w)


def _tc_last(p0, p1, t, d0, d1, b3, wc, bc):
  def body(p0_ref, p1_ref, t_ref, d0_ref, d1_ref, b_ref, wc_ref, bc_ref,
           out_ref, h_ref):
    dinv = lax.rsqrt(d0_ref[...] + d1_ref[...] + 1.0)
    acc = (p0_ref[...] + p1_ref[...] + t_ref[...])[:, :64]
    h = jnp.tanh(dinv * acc + b_ref[...])
    h_ref[...] = h
    out_ref[...] = jnp.dot(h, wc_ref[...],
                           preferred_element_type=jnp.float32) + bc_ref[...]
  return pl.pallas_call(
      body,
      grid=(_GRID,),
      in_specs=[_row_spec(128), _row_spec(128), _row_spec(128),
                _row_spec(1), _row_spec(1),
                _full_spec(1, 64), _full_spec(64, 16), _full_spec(1, 16)],
      out_specs=[_row_spec(16), _row_spec(64)],
      out_shape=[jax.ShapeDtypeStruct((N_NODES, 16), jnp.float32),
                 jax.ShapeDtypeStruct((N_NODES, 64), jnp.float32)],
  )(p0, p1, t, d0, d1, b3, wc, bc)


_sc_deg = _make_sc_pass(16, gather=False)
_sc_agg128 = _make_sc_pass(128, gather=True)


def kernel(x, edge_index, W1, b1, W2, b2, W3, b3, Wc, bc):
  ei = edge_index.astype(jnp.int32)
  npad = E_PAD - N_EDGES
  src = jnp.concatenate([ei[0], jnp.zeros((npad,), jnp.int32)])
  dst = jnp.concatenate([ei[1], jnp.full((npad,), N_NODES, jnp.int32)])
  src_r = src.reshape(NS, K_TILE, CHUNK)
  dst_r = dst.reshape(NS, K_TILE, CHUNK)
  dst_d = dst.reshape(NC, NS, K_D, CHUNK_D)

  deg_parts = _sc_deg(dst_d)
  d0 = deg_parts[0, :N_NODES, 0:1]
  d1 = deg_parts[1, :N_NODES, 0:1]

  b1r = b1.reshape(1, 128)
  b2r = b2.reshape(1, 128)
  b3r = b3.reshape(1, 64)
  bcr = bc.reshape(1, 16)

  t1 = _tc_first(x, W1, d0, d1)
  a1 = _sc_agg128(src_r, dst_r, t1)
  t2 = _tc_mid(a1[0, :N_NODES], a1[1, :N_NODES], t1, d0, d1, b1r, W2, 128)
  a2 = _sc_agg128(src_r, dst_r, t2)
  w3p = jnp.pad(W3, ((0, 0), (0, 64)))
  t3 = _tc_mid(a2[0, :N_NODES], a2[1, :N_NODES], t2, d0, d1, b2r, w3p, 128)
  a3 = _sc_agg128(src_r, dst_r, t3)
  out, h3 = _tc_last(a3[0, :N_NODES], a3[1, :N_NODES], t3, d0, d1,
                     b3r, Wc, bcr)
  return (out, h3)
